# baseline (speedup 1.0000x reference)
import functools

import jax
import jax.numpy as jnp
from jax.experimental import pallas as pl
from jax.experimental.pallas import tpu as pltpu

DIM = 768
NUM_EXPERTS = 64
TOP_K = 2
HIDDEN = DIM // 4
B = 16
HW = 1024
N_TOKENS = B * HW
NBUF = 6


def _router_kernel(x_hbm, w1_ref, c_ref, w2_ref, b2_ref,
                   probs_out_ref, idx_out_ref, loss_out_ref,
                   buf_ref, sem):
    for p in range(NBUF - 2):
        pltpu.make_async_copy(x_hbm.at[p], buf_ref.at[p], sem.at[p]).start()

    w1 = w1_ref[...]
    w2 = w2_ref[...]
    c = c_ref[...]
    b2 = b2_ref[...]
    CH = 256
    NCH = HW // CH
    iota = jax.lax.broadcasted_iota(jnp.int32, (NUM_EXPERTS, CH), 0)
    ones_row = jnp.ones((1, NUM_EXPERTS), jnp.float32)

    usage_acc = jnp.zeros((NUM_EXPERTS, 1), jnp.float32)
    ent_acc = jnp.zeros((1, 1), jnp.float32)

    def tail(b, lfull):
        nonlocal usage_acc, ent_acc
        for k in range(NCH):
            sl = slice(k * CH, (k + 1) * CH)
            logits = jnp.clip(lfull[:, sl] + b2, -10.0, 10.0)

            l1 = jnp.max(logits, axis=0, keepdims=True)
            i1 = jnp.min(jnp.where(logits == l1, iota, NUM_EXPERTS), axis=0,
                         keepdims=True)
            lm = jnp.where(iota == i1, -jnp.inf, logits)
            l2 = jnp.max(lm, axis=0, keepdims=True)
            i2 = jnp.min(jnp.where(lm == l2, iota, NUM_EXPERTS), axis=0,
                         keepdims=True)

            e = jnp.exp(logits)
            el = e * logits
            s = jnp.dot(ones_row, e,
                        preferred_element_type=jnp.float32)
            sel = jnp.dot(ones_row, el,
                          preferred_element_type=jnp.float32)
            rs = 1.0 / s
            usage_acc = usage_acc + jax.lax.dot_general(
                e, rs, (((1,), (1,)), ((), ())),
                preferred_element_type=jnp.float32)
            ent_row = jnp.log(s) - sel * rs
            ent_acc = ent_acc + jnp.sum(ent_row, axis=1, keepdims=True)

            p1 = jnp.exp(l1) * rs
            p2 = jnp.exp(l2) * rs
            rden = 1.0 / (p1 + p2 + 1e-8)
            probs_out_ref[b, :, sl] = jnp.concatenate(
                [p1 * rden, p2 * rden], axis=0)
            idx_out_ref[b, :, sl] = jnp.concatenate([i1, i2], axis=0)

    def gate(b):
        xb = buf_ref[b % NBUF]
        hfull = jnp.dot(w1, xb, preferred_element_type=jnp.float32)
        hfull = jnp.maximum(hfull + c, 0.0)
        return jnp.dot(w2, hfull, preferred_element_type=jnp.float32)

    for bp in range(B // 2):
        b0, b1 = 2 * bp, 2 * bp + 1
        for b in (b0, b1):
            if b + NBUF - 2 < B:
                nx = (b + NBUF - 2) % NBUF
                pltpu.make_async_copy(x_hbm.at[b + NBUF - 2], buf_ref.at[nx],
                                      sem.at[nx]).start()
        pltpu.make_async_copy(x_hbm.at[b0], buf_ref.at[b0 % NBUF],
                              sem.at[b0 % NBUF]).wait()
        pltpu.make_async_copy(x_hbm.at[b1], buf_ref.at[b1 % NBUF],
                              sem.at[b1 % NBUF]).wait()
        l0 = gate(b0)
        l1f = gate(b1)
        tail(b0, l0)
        tail(b1, l1f)

    usage_mean = usage_acc / N_TOKENS
    lb = jnp.sum((usage_mean - 1.0 / NUM_EXPERTS) ** 2)
    entropy = jnp.sum(ent_acc) / N_TOKENS
    coef = 1e-05 + (0.0005 - 1e-05)
    loss_out_ref[...] = jnp.reshape(lb * coef + (-entropy) * 0.001, (1, 1))


@functools.partial(jax.jit, static_argnames=())
def _run(x, w1f, c, w2, b2):
    xf = x.reshape(B, DIM, HW)
    out_shapes = (
        jax.ShapeDtypeStruct((B, TOP_K, HW), jnp.float32),
        jax.ShapeDtypeStruct((B, TOP_K, HW), jnp.int32),
        jax.ShapeDtypeStruct((1, 1), jnp.float32),
    )
    probs, idx, loss = pl.pallas_call(
        _router_kernel,
        in_specs=[
            pl.BlockSpec(memory_space=pltpu.MemorySpace.HBM),
            pl.BlockSpec(memory_space=pltpu.VMEM),
            pl.BlockSpec(memory_space=pltpu.VMEM),
            pl.BlockSpec(memory_space=pltpu.VMEM),
            pl.BlockSpec(memory_space=pltpu.VMEM),
        ],
        out_specs=(
            pl.BlockSpec(memory_space=pltpu.VMEM),
            pl.BlockSpec(memory_space=pltpu.VMEM),
            pl.BlockSpec(memory_space=pltpu.VMEM),
        ),
        out_shape=out_shapes,
        scratch_shapes=[
            pltpu.VMEM((NBUF, DIM, HW), jnp.float32),
            pltpu.SemaphoreType.DMA((NBUF,)),
        ],
    )(xf, w1f, c, w2, b2)
    return probs, idx, loss


def kernel(x, w1, b1, gamma, beta, running_mean, running_var, w2, b2):
    a = gamma * jax.lax.rsqrt(running_var + 1e-5)
    w1f = w1 * a[:, None]
    c = (b1 - running_mean) * a + beta
    probs, idx, loss = _run(
        x, w1f, c.reshape(HIDDEN, 1), w2, b2.reshape(NUM_EXPERTS, 1),
    )
    H = W = 32
    return (probs.reshape(B, TOP_K, H, W), idx.reshape(B, TOP_K, H, W),
            loss[0, 0])


# speedup vs baseline: 1.0121x; 1.0121x over previous
import functools

import jax
import jax.numpy as jnp
from jax.experimental import pallas as pl
from jax.experimental.pallas import tpu as pltpu

DIM = 768
NUM_EXPERTS = 64
TOP_K = 2
HIDDEN = DIM // 4
B = 16
HW = 1024
N_TOKENS = B * HW
NBUF = 8


def _router_kernel(x_hbm, w1_ref, a_ref, c_ref, w2_ref, b2_ref,
                   probs_out_ref, idx_out_ref, loss_out_ref,
                   buf_ref, sem):
    for p in range(NBUF - 2):
        pltpu.make_async_copy(x_hbm.at[p], buf_ref.at[p], sem.at[p]).start()

    w1 = w1_ref[...]
    w2 = w2_ref[...]
    a = a_ref[...]
    c = c_ref[...]
    b2 = b2_ref[...]
    CH = 256
    NCH = HW // CH
    iota = jax.lax.broadcasted_iota(jnp.int32, (NUM_EXPERTS, CH), 0)

    usage_acc = jnp.zeros((NUM_EXPERTS, 1), jnp.float32)
    ent_acc = jnp.zeros((1, 1), jnp.float32)

    def tail(b, lfull):
        nonlocal usage_acc, ent_acc
        for k in range(NCH):
            sl = slice(k * CH, (k + 1) * CH)
            logits = jnp.clip(lfull[:, sl] + b2, -10.0, 10.0)

            l1 = jnp.max(logits, axis=0, keepdims=True)
            i1 = jnp.min(jnp.where(logits == l1, iota, NUM_EXPERTS), axis=0,
                         keepdims=True)
            lm = jnp.where(iota == i1, -jnp.inf, logits)
            l2 = jnp.max(lm, axis=0, keepdims=True)
            i2 = jnp.min(jnp.where(lm == l2, iota, NUM_EXPERTS), axis=0,
                         keepdims=True)

            e = jnp.exp(logits)
            s = jnp.sum(e, axis=0, keepdims=True)
            sel = jnp.sum(e * logits, axis=0, keepdims=True)
            rs = 1.0 / s
            usage_acc = usage_acc + jnp.sum(e * rs, axis=1, keepdims=True)
            ent_row = jnp.log(s) - sel * rs
            ent_acc = ent_acc + jnp.sum(ent_row, axis=1, keepdims=True)

            p1 = jnp.exp(l1) * rs
            p2 = jnp.exp(l2) * rs
            rden = 1.0 / (p1 + p2 + 1e-8)
            probs_out_ref[b, :, sl] = jnp.concatenate(
                [p1 * rden, p2 * rden], axis=0)
            idx_out_ref[b, :, sl] = jnp.concatenate([i1, i2], axis=0)

    def gate(b):
        xb = buf_ref[b % NBUF]
        hfull = jnp.dot(w1, xb, preferred_element_type=jnp.float32)
        hfull = jnp.maximum(hfull * a + c, 0.0)
        return jnp.dot(w2, hfull, preferred_element_type=jnp.float32)

    for bp in range(B // 2):
        b0, b1 = 2 * bp, 2 * bp + 1
        pltpu.make_async_copy(x_hbm.at[b0], buf_ref.at[b0 % NBUF],
                              sem.at[b0 % NBUF]).wait()
        pltpu.make_async_copy(x_hbm.at[b1], buf_ref.at[b1 % NBUF],
                              sem.at[b1 % NBUF]).wait()
        for b in (b0, b1):
            if b + NBUF - 2 < B:
                nx = (b + NBUF - 2) % NBUF
                pltpu.make_async_copy(x_hbm.at[b + NBUF - 2], buf_ref.at[nx],
                                      sem.at[nx]).start()
        l0 = gate(b0)
        l1f = gate(b1)
        tail(b0, l0)
        tail(b1, l1f)

    usage_mean = usage_acc / N_TOKENS
    lb = jnp.sum((usage_mean - 1.0 / NUM_EXPERTS) ** 2)
    entropy = jnp.sum(ent_acc) / N_TOKENS
    coef = 1e-05 + (0.0005 - 1e-05)
    loss_out_ref[...] = jnp.reshape(lb * coef + (-entropy) * 0.001, (1, 1))


@functools.partial(jax.jit, static_argnames=())
def _run(x, w1, a, c, w2, b2):
    xf = x.reshape(B, DIM, HW)
    out_shapes = (
        jax.ShapeDtypeStruct((B, TOP_K, HW), jnp.float32),
        jax.ShapeDtypeStruct((B, TOP_K, HW), jnp.int32),
        jax.ShapeDtypeStruct((1, 1), jnp.float32),
    )
    probs, idx, loss = pl.pallas_call(
        _router_kernel,
        in_specs=[
            pl.BlockSpec(memory_space=pltpu.MemorySpace.HBM),
            pl.BlockSpec(memory_space=pltpu.VMEM),
            pl.BlockSpec(memory_space=pltpu.VMEM),
            pl.BlockSpec(memory_space=pltpu.VMEM),
            pl.BlockSpec(memory_space=pltpu.VMEM),
            pl.BlockSpec(memory_space=pltpu.VMEM),
        ],
        out_specs=(
            pl.BlockSpec(memory_space=pltpu.VMEM),
            pl.BlockSpec(memory_space=pltpu.VMEM),
            pl.BlockSpec(memory_space=pltpu.VMEM),
        ),
        out_shape=out_shapes,
        scratch_shapes=[
            pltpu.VMEM((NBUF, DIM, HW), jnp.float32),
            pltpu.SemaphoreType.DMA((NBUF,)),
        ],
    )(xf, w1, a, c, w2, b2)
    return probs, idx, loss


def kernel(x, w1, b1, gamma, beta, running_mean, running_var, w2, b2):
    a = gamma * jax.lax.rsqrt(running_var + 1e-5)
    c = (b1 - running_mean) * a + beta
    probs, idx, loss = _run(
        x, w1, a.reshape(HIDDEN, 1), c.reshape(HIDDEN, 1), w2,
        b2.reshape(NUM_EXPERTS, 1),
    )
    H = W = 32
    return (probs.reshape(B, TOP_K, H, W), idx.reshape(B, TOP_K, H, W),
            loss[0, 0])
